# Initial kernel scaffold; baseline (speedup 1.0000x reference)
import jax
import jax.numpy as jnp
from jax.experimental import pallas as pl


def kernel(features, edges, W1, b1, W2, b2):
    raise NotImplementedError("write your pallas kernel here")


# speedup vs baseline: 20.4008x; 20.4008x over previous
import functools

import jax
import jax.numpy as jnp
from jax import lax
from jax.experimental import pallas as pl
from jax.experimental.pallas import tpu as pltpu
from jax.experimental.pallas import tpu_sc as plsc

N = 10000
E = 320000
DF = 128
DH = 128
DC = 64

NC = 2
NS = 16
NW = NC * NS

NPAD = 10240
RT = NPAD // NS

CH = 125
EPT = E // NW
NCHUNK = EPT // CH
EROWS = E // CH
ERT = EROWS // NW

R = 256
GRID = NPAD // R

_mesh = plsc.VectorSubcoreMesh(core_axis_name="c", subcore_axis_name="s")


@functools.partial(
    pl.kernel,
    out_type=jax.ShapeDtypeStruct((NC, NPAD), jnp.float32),
    mesh=_mesh,
    scratch_types=[
        pltpu.VMEM((ERT, CH), jnp.int32),
        pltpu.VMEM((128,), jnp.float32),
        pltpu.VMEM((RT,), jnp.float32),
        pltpu.VMEM_SHARED((NPAD,), jnp.float32),
    ],
)
def _deg_kernel(dst_hbm, out_hbm, dst_v, ones_v, zeros_v, acc_sh):
    c = lax.axis_index("c")
    s = lax.axis_index("s")
    wid = c * NS + s
    for k in range(RT // 16):
        zeros_v[pl.ds(k * 16, 16)] = jnp.zeros((16,), jnp.float32)
    for k in range(128 // 16):
        ones_v[pl.ds(k * 16, 16)] = jnp.ones((16,), jnp.float32)
    pltpu.sync_copy(zeros_v, acc_sh.at[pl.ds(s * RT, RT)])
    pltpu.sync_copy(dst_hbm.at[pl.ds(wid * ERT, ERT)], dst_v)
    plsc.subcore_barrier()

    def body(i, carry):
        pltpu.sync_copy(ones_v.at[pl.ds(0, CH)], acc_sh.at[dst_v.at[i]], add=True)
        return carry

    lax.fori_loop(0, ERT, body, 0)
    plsc.subcore_barrier()
    pltpu.sync_copy(acc_sh.at[pl.ds(s * RT, RT)], out_hbm.at[c, pl.ds(s * RT, RT)])


def _make_agg(d_feat):
    @functools.partial(
        pl.kernel,
        out_type=jax.ShapeDtypeStruct((NC, NPAD, d_feat), jnp.float32),
        mesh=_mesh,
        scratch_types=[
            pltpu.VMEM((ERT, CH), jnp.int32),
            pltpu.VMEM((ERT, CH), jnp.int32),
            pltpu.VMEM((CH, d_feat), jnp.float32),
            pltpu.VMEM_SHARED((NPAD, d_feat), jnp.float32),
            pltpu.SemaphoreType.DMA,
        ],
    )
    def _agg_kernel(h_hbm, src_hbm, dst_hbm, zeros_hbm, out_hbm,
                    src_v, dst_v, rows_v, acc_sh, sem):
        c = lax.axis_index("c")
        s = lax.axis_index("s")
        wid = c * NS + s
        pltpu.sync_copy(zeros_hbm.at[pl.ds(s * RT, RT)],
                        acc_sh.at[pl.ds(s * RT, RT)])
        pltpu.sync_copy(src_hbm.at[pl.ds(wid * ERT, ERT)], src_v)
        pltpu.sync_copy(dst_hbm.at[pl.ds(wid * ERT, ERT)], dst_v)
        plsc.subcore_barrier()

        def body(i, carry):
            pltpu.async_copy(h_hbm.at[src_v.at[i]], rows_v, sem).wait()
            pltpu.sync_copy(rows_v, acc_sh.at[dst_v.at[i]], add=True)
            return carry

        lax.fori_loop(0, ERT, body, 0)
        plsc.subcore_barrier()
        pltpu.sync_copy(acc_sh.at[pl.ds(s * RT, RT)],
                        out_hbm.at[c, pl.ds(s * RT, RT)])

    return _agg_kernel


_agg128 = _make_agg(DH)


def _dinv_col(degp_ref):
    dsum = degp_ref[0, :] + degp_ref[1, :] + 1.0
    dl = lax.rsqrt(dsum)
    rows = lax.broadcasted_iota(jnp.int32, (R, R), 0)
    cols = lax.broadcasted_iota(jnp.int32, (R, R), 1)
    diag = jnp.where(rows == cols, dl[None, :], 0.0)
    return jnp.sum(diag, axis=1, keepdims=True)


def _mm1_body(x_ref, w_ref, degp_ref, h_ref, dinv_ref):
    col = _dinv_col(degp_ref)
    h = jnp.dot(x_ref[...], w_ref[...], preferred_element_type=jnp.float32)
    h_ref[...] = h * col
    dinv_ref[...] = col


def _mm2_body(aggp_ref, h1_ref, dinv_ref, b1_ref, w2_ref, h2_ref):
    agg = aggp_ref[0] + aggp_ref[1]
    z = dinv_ref[...] * (agg + h1_ref[...]) + b1_ref[...]
    z = jnp.maximum(z, 0.0)
    h2 = jnp.dot(z, w2_ref[...], preferred_element_type=jnp.float32)
    h2_ref[...] = jnp.concatenate(
        [h2 * dinv_ref[...], jnp.zeros((R, DH - DC), jnp.float32)], axis=1)


def _out_body(aggp_ref, h2_ref, dinv_ref, b2_ref, out_ref):
    acc = aggp_ref[0] + aggp_ref[1] + h2_ref[...]
    z = dinv_ref[...] * acc[:, :DC] + b2_ref[...]
    m = jnp.max(z, axis=1, keepdims=True)
    lse = jnp.log(jnp.sum(jnp.exp(z - m), axis=1, keepdims=True))
    out_ref[...] = z - m - lse


def _tc_layer1(xp, W1, degp):
    return pl.pallas_call(
        _mm1_body,
        grid=(GRID,),
        in_specs=[
            pl.BlockSpec((R, DF), lambda i: (i, 0)),
            pl.BlockSpec((DF, DH), lambda i: (0, 0)),
            pl.BlockSpec((NC, R), lambda i: (0, i)),
        ],
        out_specs=[
            pl.BlockSpec((R, DH), lambda i: (i, 0)),
            pl.BlockSpec((R, 1), lambda i: (i, 0)),
        ],
        out_shape=[
            jax.ShapeDtypeStruct((NPAD, DH), jnp.float32),
            jax.ShapeDtypeStruct((NPAD, 1), jnp.float32),
        ],
    )(xp, W1, degp)


def _tc_layer2(agg1, h1p, dinvc, b1, W2):
    return pl.pallas_call(
        _mm2_body,
        grid=(GRID,),
        in_specs=[
            pl.BlockSpec((NC, R, DH), lambda i: (0, i, 0)),
            pl.BlockSpec((R, DH), lambda i: (i, 0)),
            pl.BlockSpec((R, 1), lambda i: (i, 0)),
            pl.BlockSpec((1, DH), lambda i: (0, 0)),
            pl.BlockSpec((DH, DC), lambda i: (0, 0)),
        ],
        out_specs=pl.BlockSpec((R, DH), lambda i: (i, 0)),
        out_shape=jax.ShapeDtypeStruct((NPAD, DH), jnp.float32),
    )(agg1, h1p, dinvc, b1, W2)


def _tc_out(agg2, h2p, dinvc, b2):
    return pl.pallas_call(
        _out_body,
        grid=(GRID,),
        in_specs=[
            pl.BlockSpec((NC, R, DH), lambda i: (0, i, 0)),
            pl.BlockSpec((R, DH), lambda i: (i, 0)),
            pl.BlockSpec((R, 1), lambda i: (i, 0)),
            pl.BlockSpec((1, DC), lambda i: (0, 0)),
        ],
        out_specs=pl.BlockSpec((R, DC), lambda i: (i, 0)),
        out_shape=jax.ShapeDtypeStruct((NPAD, DC), jnp.float32),
    )(agg2, h2p, dinvc, b2)


def kernel(features, edges, W1, b1, W2, b2):
    src = edges[0].astype(jnp.int32).reshape(EROWS, CH)
    dst = edges[1].astype(jnp.int32).reshape(EROWS, CH)
    xp = jnp.pad(features, ((0, NPAD - N), (0, 0)))

    degp = _deg_kernel(dst)
    h1p, dinvc = _tc_layer1(xp, W1, degp)
    agg1 = _agg128(h1p, src, dst, jnp.zeros((NPAD, DH), jnp.float32))
    h2p = _tc_layer2(agg1, h1p, dinvc, b1.reshape(1, DH), W2)
    agg2 = _agg128(h2p, src, dst, jnp.zeros((NPAD, DH), jnp.float32))
    outp = _tc_out(agg2, h2p, dinvc, b2.reshape(1, DC))
    return outp[:N]


# speedup vs baseline: 26.3185x; 1.2901x over previous
import functools

import jax
import jax.numpy as jnp
from jax import lax
from jax.experimental import pallas as pl
from jax.experimental.pallas import tpu as pltpu
from jax.experimental.pallas import tpu_sc as plsc

N = 10000
E = 320000
DF = 128
DH = 128
DC = 64

NC = 2
NS = 16
NW = NC * NS

NPAD = 10240
RT = NPAD // NS

CH = 80
EPT = E // NW
NCHUNK = EPT // CH

R = 256
GRID = NPAD // R

_mesh = plsc.VectorSubcoreMesh(core_axis_name="c", subcore_axis_name="s")


@functools.partial(
    pl.kernel,
    out_type=jax.ShapeDtypeStruct((NC, NPAD), jnp.float32),
    mesh=_mesh,
    scratch_types=[
        pltpu.VMEM((EPT,), jnp.int32),
        pltpu.VMEM((CH,), jnp.float32),
        pltpu.VMEM((RT,), jnp.float32),
        pltpu.VMEM_SHARED((NPAD,), jnp.float32),
    ],
)
def _deg_kernel(dst_hbm, out_hbm, dst_v, ones_v, zeros_v, acc_sh):
    c = lax.axis_index("c")
    s = lax.axis_index("s")
    wid = c * NS + s
    for k in range(RT // 16):
        zeros_v[pl.ds(k * 16, 16)] = jnp.zeros((16,), jnp.float32)
    for k in range(CH // 16):
        ones_v[pl.ds(k * 16, 16)] = jnp.ones((16,), jnp.float32)
    pltpu.sync_copy(zeros_v, acc_sh.at[pl.ds(s * RT, RT)])
    pltpu.sync_copy(dst_hbm.at[pl.ds(wid * EPT, EPT)], dst_v)
    plsc.subcore_barrier()

    def body(i, carry):
        pltpu.sync_copy(ones_v, acc_sh.at[dst_v.at[pl.ds(i * CH, CH)]],
                        add=True)
        return carry

    lax.fori_loop(0, NCHUNK, body, 0)
    plsc.subcore_barrier()
    pltpu.sync_copy(acc_sh.at[pl.ds(s * RT, RT)], out_hbm.at[c, pl.ds(s * RT, RT)])


def _make_agg(d_feat):
    @functools.partial(
        pl.kernel,
        out_type=jax.ShapeDtypeStruct((NC, NPAD, d_feat), jnp.float32),
        mesh=_mesh,
        scratch_types=[
            pltpu.VMEM((EPT,), jnp.int32),
            pltpu.VMEM((EPT,), jnp.int32),
            pltpu.VMEM((CH, d_feat), jnp.float32),
            pltpu.VMEM((CH, d_feat), jnp.float32),
            pltpu.VMEM_SHARED((NPAD, d_feat), jnp.float32),
            pltpu.SemaphoreType.DMA,
            pltpu.SemaphoreType.DMA,
        ],
    )
    def _agg_kernel(h_hbm, src_hbm, dst_hbm, zeros_hbm, out_hbm,
                    src_v, dst_v, rows_a, rows_b, acc_sh, sem_a, sem_b):
        c = lax.axis_index("c")
        s = lax.axis_index("s")
        wid = c * NS + s
        pltpu.sync_copy(zeros_hbm.at[pl.ds(s * RT, RT)],
                        acc_sh.at[pl.ds(s * RT, RT)])
        pltpu.sync_copy(src_hbm.at[pl.ds(wid * EPT, EPT)], src_v)
        pltpu.sync_copy(dst_hbm.at[pl.ds(wid * EPT, EPT)], dst_v)
        plsc.subcore_barrier()

        def _gather(k, buf, sem):
            return pltpu.async_copy(
                h_hbm.at[src_v.at[pl.ds(k * CH, CH)]], buf, sem)

        def _scatter(k, buf):
            pltpu.sync_copy(buf, acc_sh.at[dst_v.at[pl.ds(k * CH, CH)]],
                            add=True)

        _gather(0, rows_a, sem_a)

        def body(j, carry):
            a = 2 * j
            b = a + 1
            _gather(b, rows_b, sem_b)
            pltpu.make_async_copy(
                h_hbm.at[src_v.at[pl.ds(a * CH, CH)]], rows_a, sem_a).wait()
            _scatter(a, rows_a)

            @pl.when(j < NCHUNK // 2 - 1)
            def _start_next_a():
                _gather(a + 2, rows_a, sem_a)

            pltpu.make_async_copy(
                h_hbm.at[src_v.at[pl.ds(b * CH, CH)]], rows_b, sem_b).wait()
            _scatter(b, rows_b)
            return carry

        lax.fori_loop(0, NCHUNK // 2, body, 0)
        if NCHUNK % 2:
            _gather(NCHUNK - 1, rows_a, sem_a).wait()
            _scatter(NCHUNK - 1, rows_a)
        plsc.subcore_barrier()
        pltpu.sync_copy(acc_sh.at[pl.ds(s * RT, RT)],
                        out_hbm.at[c, pl.ds(s * RT, RT)])

    return _agg_kernel


_agg128 = _make_agg(DH)


def _dinv_col(degp_ref):
    dsum = degp_ref[0, :] + degp_ref[1, :] + 1.0
    dl = lax.rsqrt(dsum)
    rows = lax.broadcasted_iota(jnp.int32, (R, R), 0)
    cols = lax.broadcasted_iota(jnp.int32, (R, R), 1)
    diag = jnp.where(rows == cols, dl[None, :], 0.0)
    return jnp.sum(diag, axis=1, keepdims=True)


def _mm1_body(x_ref, w_ref, degp_ref, h_ref, dinv_ref):
    col = _dinv_col(degp_ref)
    h = jnp.dot(x_ref[...], w_ref[...], preferred_element_type=jnp.float32)
    h_ref[...] = h * col
    dinv_ref[...] = col


def _mm2_body(aggp_ref, h1_ref, dinv_ref, b1_ref, w2_ref, h2_ref):
    agg = aggp_ref[0] + aggp_ref[1]
    z = dinv_ref[...] * (agg + h1_ref[...]) + b1_ref[...]
    z = jnp.maximum(z, 0.0)
    h2 = jnp.dot(z, w2_ref[...], preferred_element_type=jnp.float32)
    h2_ref[...] = jnp.concatenate(
        [h2 * dinv_ref[...], jnp.zeros((R, DH - DC), jnp.float32)], axis=1)


def _out_body(aggp_ref, h2_ref, dinv_ref, b2_ref, out_ref):
    acc = aggp_ref[0] + aggp_ref[1] + h2_ref[...]
    z = dinv_ref[...] * acc[:, :DC] + b2_ref[...]
    m = jnp.max(z, axis=1, keepdims=True)
    lse = jnp.log(jnp.sum(jnp.exp(z - m), axis=1, keepdims=True))
    out_ref[...] = z - m - lse


def _tc_layer1(xp, W1, degp):
    return pl.pallas_call(
        _mm1_body,
        grid=(GRID,),
        in_specs=[
            pl.BlockSpec((R, DF), lambda i: (i, 0)),
            pl.BlockSpec((DF, DH), lambda i: (0, 0)),
            pl.BlockSpec((NC, R), lambda i: (0, i)),
        ],
        out_specs=[
            pl.BlockSpec((R, DH), lambda i: (i, 0)),
            pl.BlockSpec((R, 1), lambda i: (i, 0)),
        ],
        out_shape=[
            jax.ShapeDtypeStruct((NPAD, DH), jnp.float32),
            jax.ShapeDtypeStruct((NPAD, 1), jnp.float32),
        ],
    )(xp, W1, degp)


def _tc_layer2(agg1, h1p, dinvc, b1, W2):
    return pl.pallas_call(
        _mm2_body,
        grid=(GRID,),
        in_specs=[
            pl.BlockSpec((NC, R, DH), lambda i: (0, i, 0)),
            pl.BlockSpec((R, DH), lambda i: (i, 0)),
            pl.BlockSpec((R, 1), lambda i: (i, 0)),
            pl.BlockSpec((1, DH), lambda i: (0, 0)),
            pl.BlockSpec((DH, DC), lambda i: (0, 0)),
        ],
        out_specs=pl.BlockSpec((R, DH), lambda i: (i, 0)),
        out_shape=jax.ShapeDtypeStruct((NPAD, DH), jnp.float32),
    )(agg1, h1p, dinvc, b1, W2)


def _tc_out(agg2, h2p, dinvc, b2):
    return pl.pallas_call(
        _out_body,
        grid=(GRID,),
        in_specs=[
            pl.BlockSpec((NC, R, DH), lambda i: (0, i, 0)),
            pl.BlockSpec((R, DH), lambda i: (i, 0)),
            pl.BlockSpec((R, 1), lambda i: (i, 0)),
            pl.BlockSpec((1, DC), lambda i: (0, 0)),
        ],
        out_specs=pl.BlockSpec((R, DC), lambda i: (i, 0)),
        out_shape=jax.ShapeDtypeStruct((NPAD, DC), jnp.float32),
    )(agg2, h2p, dinvc, b2)


def kernel(features, edges, W1, b1, W2, b2):
    src = edges[0].astype(jnp.int32)
    dst = edges[1].astype(jnp.int32)
    xp = jnp.pad(features, ((0, NPAD - N), (0, 0)))

    degp = _deg_kernel(dst)
    h1p, dinvc = _tc_layer1(xp, W1, degp)
    agg1 = _agg128(h1p, src, dst, jnp.zeros((NPAD, DH), jnp.float32))
    h2p = _tc_layer2(agg1, h1p, dinvc, b1.reshape(1, DH), W2)
    agg2 = _agg128(h2p, src, dst, jnp.zeros((NPAD, DH), jnp.float32))
    outp = _tc_out(agg2, h2p, dinvc, b2.reshape(1, DC))
    return outp[:N]
